# baseline (speedup 1.0000x reference)
import functools

import jax
import jax.numpy as jnp
from jax import lax
from jax.experimental import pallas as pl
from jax.experimental.pallas import tpu as pltpu
from jax.experimental.pallas import tpu_sc as plsc

N = 10000
E = 320000
NT = 16
NC = 2
CH = 64
BK = 32
NBI = 10
CPT = NBI * BK
EPT = CPT * CH
EPAD = NT * EPT
AGG_ROWS = 10048
DEG_ROWS = 10240
RPT = 640
RPT_LAST = N - (NT - 1) * RPT
R = 2000
NBLK = (2 * N) // R

@functools.cache
def _get_mesh():
    return plsc.VectorSubcoreMesh(core_axis_name="c", subcore_axis_name="s",
                                  num_cores=NC, num_subcores=NT)


def _deg_body(dsts_hbm, out_hbm, ones_v, dst_v, acc_sh):
    cid = lax.axis_index("c")
    sid = lax.axis_index("s")

    def _fill(i, carry):
        ones_v[i] = jnp.ones((16,), jnp.float32)
        return carry

    lax.fori_loop(0, CH, _fill, 0)
    for k in range(DEG_ROWS // NT // CH):
        pltpu.sync_copy(ones_v, acc_sh.at[pl.ds(sid * (DEG_ROWS // NT) + k * CH, CH)])
    pltpu.sync_copy(dsts_hbm.at[cid, sid], dst_v)
    plsc.subcore_barrier()

    def _blk(b, carry):
        for k in range(BK):
            pltpu.sync_copy(ones_v, acc_sh.at[dst_v.at[b, k]], add=True)
        return carry

    lax.fori_loop(0, NBI, _blk, 0)
    plsc.subcore_barrier()

    @pl.when(sid < NT - 1)
    def _():
        pltpu.sync_copy(acc_sh.at[pl.ds(sid * RPT, RPT)],
                        out_hbm.at[pl.ds(cid * N + sid * RPT, RPT)])

    @pl.when(sid == NT - 1)
    def _():
        pltpu.sync_copy(acc_sh.at[pl.ds((NT - 1) * RPT, RPT_LAST)],
                        out_hbm.at[pl.ds(cid * N + (NT - 1) * RPT, RPT_LAST)])


@functools.cache
def _deg_kernel_fn():
    return pl.kernel(
        _deg_body,
        out_type=jax.ShapeDtypeStruct((2 * N, 16), jnp.float32),
        mesh=_get_mesh(),
        scratch_types=[
            pltpu.VMEM((CH, 16), jnp.float32),
            pltpu.VMEM((NBI, BK, CH), jnp.int32),
            pltpu.VMEM_SHARED((DEG_ROWS, 16), jnp.float32),
        ],
    )


def _deg_kernel(dsts):
    return _deg_kernel_fn()(dsts)


def _agg_body(h_hbm, srcs_hbm, dsts_hbm, out_hbm,
              sidx, didx, rows_v, acc_sh, sem_i, sem_g, sem_s):
    cid = lax.axis_index("c")
    sid = lax.axis_index("s")
    @pl.when(sid < NT - 1)
    def _():
        pltpu.sync_copy(h_hbm.at[pl.ds(cid * N + sid * RPT, RPT)],
                        acc_sh.at[pl.ds(sid * RPT, RPT)])

    @pl.when(sid == NT - 1)
    def _():
        pltpu.sync_copy(h_hbm.at[pl.ds(cid * N + (NT - 1) * RPT, RPT_LAST)],
                        acc_sh.at[pl.ds((NT - 1) * RPT, RPT_LAST)])

    pltpu.sync_copy(srcs_hbm.at[cid, sid, 0], sidx.at[0])
    pltpu.sync_copy(dsts_hbm.at[cid, sid, 0], didx.at[0])
    plsc.subcore_barrier()
    pltpu.async_copy(h_hbm.at[sidx.at[0, 0]], rows_v.at[0], sem_g.at[0])
    pltpu.async_copy(h_hbm.at[sidx.at[0, 1]], rows_v.at[1], sem_g.at[1])

    def _blk(b, carry):
        islot = lax.rem(b, 2)

        @pl.when(b + 1 < NBI)
        def _():
            pltpu.async_copy(srcs_hbm.at[cid, sid, b + 1],
                             sidx.at[1 - islot], sem_i)
            pltpu.async_copy(dsts_hbm.at[cid, sid, b + 1],
                             didx.at[1 - islot], sem_i)

        def _chunk(k, carry2):
            c = b * BK + k

            @pl.when(jnp.logical_and(k == BK - 2, b + 1 < NBI))
            def _():
                pltpu.make_async_copy(srcs_hbm.at[cid, sid, b + 1],
                                      sidx.at[1 - islot], sem_i).wait()
                pltpu.make_async_copy(dsts_hbm.at[cid, sid, b + 1],
                                      didx.at[1 - islot], sem_i).wait()

            c2 = c + 2
            slot = lax.rem(c, 3)
            slot2 = lax.rem(c2, 3)
            is2 = lax.rem(c2 // BK, 2)
            row2 = lax.rem(c2, BK)

            @pl.when(c2 < CPT)
            def _():
                @pl.when(c >= 1)
                def _():
                    pltpu.make_async_copy(
                        rows_v.at[slot2], acc_sh.at[didx.at[is2, row2]],
                        sem_s.at[slot2]).wait()

                pltpu.async_copy(h_hbm.at[sidx.at[is2, row2]],
                                 rows_v.at[slot2], sem_g.at[slot2])

            pltpu.make_async_copy(h_hbm.at[sidx.at[islot, k]],
                                  rows_v.at[slot], sem_g.at[slot]).wait()
            pltpu.async_copy(rows_v.at[slot], acc_sh.at[didx.at[islot, k]],
                             sem_s.at[slot], add=True)
            return carry2

        lax.fori_loop(0, BK, _chunk, 0)
        return carry

    lax.fori_loop(0, NBI, _blk, 0)
    for c in (CPT - 3, CPT - 2, CPT - 1):
        slot = c % 3
        pltpu.make_async_copy(rows_v.at[slot],
                              acc_sh.at[didx.at[(c // BK) % 2, c % BK]],
                              sem_s.at[slot]).wait()
    plsc.subcore_barrier()

    @pl.when(sid < NT - 1)
    def _():
        pltpu.sync_copy(acc_sh.at[pl.ds(sid * RPT, RPT)],
                        out_hbm.at[pl.ds(cid * N + sid * RPT, RPT)])

    @pl.when(sid == NT - 1)
    def _():
        pltpu.sync_copy(acc_sh.at[pl.ds((NT - 1) * RPT, RPT_LAST)],
                        out_hbm.at[pl.ds(cid * N + (NT - 1) * RPT, RPT_LAST)])


@functools.cache
def _agg_kernel_fn(F):
    return pl.kernel(
        _agg_body,
        out_type=jax.ShapeDtypeStruct((2 * N, F), jnp.float32),
        mesh=_get_mesh(),
        scratch_types=[
            pltpu.VMEM((2, BK, CH), jnp.int32),
            pltpu.VMEM((2, BK, CH), jnp.int32),
            pltpu.VMEM((3, CH, F), jnp.float32),
            pltpu.VMEM_SHARED((AGG_ROWS, F), jnp.float32),
            pltpu.SemaphoreType.DMA,
            pltpu.SemaphoreType.DMA((3,)),
            pltpu.SemaphoreType.DMA((3,)),
        ],
        compiler_params=pltpu.CompilerParams(use_tc_tiling_on_sc=False)
        if F < 128 else None,
    )


def _agg_kernel(F):
    return _agg_kernel_fn(F)


def _k1a_body(x_ref, w_ref, out_ref):
    out_ref[...] = jnp.dot(x_ref[...], w_ref[...],
                           preferred_element_type=jnp.float32)


def _k1b_body(h_ref, deg_ref, out_ref):
    dinv = lax.rsqrt(deg_ref[...][:, :1])
    out_ref[...] = h_ref[...] * dinv


def _layer_body(acc_ref, deg_ref, b_ref, w_ref, f_ref, h_ref, cs_ref):
    i = pl.program_id(0)
    dinv = lax.rsqrt(deg_ref[...][:, :1])
    f = acc_ref[...] * dinv + b_ref[...]
    f_ref[...] = f
    r = jnp.maximum(f, 0.0)
    h_ref[...] = jnp.dot(r, w_ref[...], preferred_element_type=jnp.float32) * dinv

    @pl.when(i % (NBLK // 2) == 0)
    def _():
        cs_ref[...] = jnp.zeros_like(cs_ref)

    cs_ref[...] += jnp.sum(f, axis=0)[None, None, :]


def _last_body(acc_ref, deg_ref, b_ref, f_ref, cs_ref):
    i = pl.program_id(0)
    dinv = lax.rsqrt(deg_ref[...][:, :1])
    f = acc_ref[...] * dinv + b_ref[...]
    f_ref[...] = f

    @pl.when(i % (NBLK // 2) == 0)
    def _():
        cs_ref[...] = jnp.zeros_like(cs_ref)

    cs_ref[...] += jnp.sum(f, axis=0)[None, None, :]


def _attn_body(f_ref, cs_ref, wa_ref, p_ref):
    i = pl.program_id(0)
    gc = jnp.tanh(jnp.dot(cs_ref[0] * (1.0 / N), wa_ref[...],
                          preferred_element_type=jnp.float32))
    f = f_ref[...]
    s = jax.nn.sigmoid(
        lax.dot_general(f, gc, (((1,), (1,)), ((), ())),
                        preferred_element_type=jnp.float32))
    contrib = lax.dot_general(s, f, (((0,), (0,)), ((), ())),
                              preferred_element_type=jnp.float32)

    @pl.when(i % (NBLK // 2) == 0)
    def _():
        p_ref[...] = jnp.zeros_like(p_ref)

    p_ref[...] += contrib[None]


def _ntn_a_body(p1_ref, p2_ref, p3_ref, t1_ref, t2_ref, t3_ref,
                o1_ref, o2_ref, o3_ref):
    for p_ref, t_ref, o_ref in ((p1_ref, t1_ref, o1_ref),
                                (p2_ref, t2_ref, o2_ref),
                                (p3_ref, t3_ref, o3_ref)):
        o_ref[...] = jnp.dot(p_ref[0], t_ref[...],
                             preferred_element_type=jnp.float32)


def _ntn_b_body(m1_ref, m2_ref, m3_ref, p1_ref, p2_ref, p3_ref,
                tb1_ref, tb2_ref, tb3_ref, tc1_ref, tc2_ref, tc3_ref,
                ws1_ref, bs1_ref, ws2_ref, bs2_ref, out_ref):
    parts = []
    for p_ref, m_ref, tbt_ref, tcr_ref in (
            (p1_ref, m1_ref, tb1_ref, tc1_ref),
            (p2_ref, m2_ref, tb2_ref, tc2_ref),
            (p3_ref, m3_ref, tb3_ref, tc3_ref)):
        e1 = p_ref[0]
        e2 = p_ref[1]
        scoring = jnp.dot(e2, m_ref[...], preferred_element_type=jnp.float32)
        comb = jnp.concatenate([e1, e2], axis=1)
        blk = jnp.dot(comb, tbt_ref[...], preferred_element_type=jnp.float32)
        parts.append(jnp.maximum(scoring + blk + tcr_ref[...], 0.0))
    scores = jnp.concatenate(parts, axis=1)
    h = jnp.maximum(jnp.dot(scores, ws1_ref[...],
                            preferred_element_type=jnp.float32) + bs1_ref[...], 0.0)
    out_ref[...] = jax.nn.sigmoid(
        jnp.dot(h, ws2_ref[...], preferred_element_type=jnp.float32) + bs2_ref[...])


def _row_spec(F):
    return pl.BlockSpec((R, F), lambda i: (i, 0))


def _full_spec(shape):
    nd = len(shape)
    return pl.BlockSpec(shape, lambda i, _n=nd: (0,) * _n)


def _cs_spec(F):
    return pl.BlockSpec((1, 1, F), lambda i: (i // (NBLK // 2), 0, 0))


def _tc_k1a(x_st, w1):
    return pl.pallas_call(
        _k1a_body,
        grid=(NBLK,),
        in_specs=[_row_spec(128), _full_spec((128, 128))],
        out_specs=_row_spec(128),
        out_shape=jax.ShapeDtypeStruct((2 * N, 128), jnp.float32),
    )(x_st, w1)


def _tc_k1b(h_raw, deg_st):
    return pl.pallas_call(
        _k1b_body,
        grid=(NBLK,),
        in_specs=[_row_spec(128), _row_spec(16)],
        out_specs=_row_spec(128),
        out_shape=jax.ShapeDtypeStruct((2 * N, 128), jnp.float32),
    )(h_raw, deg_st)


def _tc_layer(acc_st, deg_st, b_row, w_next, Fi, Fo):
    return pl.pallas_call(
        _layer_body,
        grid=(NBLK,),
        in_specs=[_row_spec(Fi), _row_spec(16), _full_spec((1, Fi)),
                  _full_spec((Fi, Fo))],
        out_specs=[_row_spec(Fi), _row_spec(Fo), _cs_spec(Fi)],
        out_shape=[jax.ShapeDtypeStruct((2 * N, Fi), jnp.float32),
                   jax.ShapeDtypeStruct((2 * N, Fo), jnp.float32),
                   jax.ShapeDtypeStruct((NC, 1, Fi), jnp.float32)],
    )(acc_st, deg_st, b_row, w_next)


def _tc_last(acc_st, deg_st, b_row, Fi):
    return pl.pallas_call(
        _last_body,
        grid=(NBLK,),
        in_specs=[_row_spec(Fi), _row_spec(16), _full_spec((1, Fi))],
        out_specs=[_row_spec(Fi), _cs_spec(Fi)],
        out_shape=[jax.ShapeDtypeStruct((2 * N, Fi), jnp.float32),
                   jax.ShapeDtypeStruct((NC, 1, Fi), jnp.float32)],
    )(acc_st, deg_st, b_row)


def _tc_attn1(f, cs, wa, F):
    return pl.pallas_call(
        _attn_body,
        grid=(NBLK,),
        in_specs=[_row_spec(F), _cs_spec(F), _full_spec((F, F))],
        out_specs=_cs_spec(F),
        out_shape=jax.ShapeDtypeStruct((NC, 1, F), jnp.float32),
    )(f, cs, wa)


def _tc_ntn_a(p1, p2, p3, t1f, t2f, t3f):
    return pl.pallas_call(
        _ntn_a_body,
        grid=(1,),
        in_specs=[_full_spec((NC, 1, 128)), _full_spec((NC, 1, 64)),
                  _full_spec((NC, 1, 32)),
                  _full_spec((128, 128 * 128)), _full_spec((64, 64 * 64)),
                  _full_spec((32, 32 * 32))],
        out_specs=[_full_spec((1, 128 * 128)), _full_spec((1, 64 * 64)),
                   _full_spec((1, 32 * 32))],
        out_shape=[jax.ShapeDtypeStruct((1, 128 * 128), jnp.float32),
                   jax.ShapeDtypeStruct((1, 64 * 64), jnp.float32),
                   jax.ShapeDtypeStruct((1, 32 * 32), jnp.float32)],
    )(p1, p2, p3, t1f, t2f, t3f)


def _tc_ntn_b(m1, m2, m3, p1, p2, p3, tb1t, tb2t, tb3t, tc1r, tc2r, tc3r,
              ws1, bs1r, ws2, bs2r):
    return pl.pallas_call(
        _ntn_b_body,
        grid=(1,),
        in_specs=[_full_spec((128, 128)), _full_spec((64, 64)),
                  _full_spec((32, 32)),
                  _full_spec((NC, 1, 128)), _full_spec((NC, 1, 64)),
                  _full_spec((NC, 1, 32)),
                  _full_spec((256, 128)), _full_spec((128, 64)),
                  _full_spec((64, 32)),
                  _full_spec((1, 128)), _full_spec((1, 64)), _full_spec((1, 32)),
                  _full_spec((224, 16)), _full_spec((1, 16)),
                  _full_spec((16, 1)), _full_spec((1, 1))],
        out_specs=_full_spec((1, 1)),
        out_shape=jax.ShapeDtypeStruct((1, 1), jnp.float32),
    )(m1, m2, m3, p1, p2, p3, tb1t, tb2t, tb3t, tc1r, tc2r, tc3r,
      ws1, bs1r, ws2, bs2r)


def _pad_edges(ei, g):
    npad = EPAD - E
    ar = jnp.arange(npad, dtype=jnp.int32)
    flat = ei.reshape(2 * E)
    src = jnp.concatenate([flat[:E] + g * N, ar % 256 + g * N])
    dst = jnp.concatenate([flat[E:], N + ar % (AGG_ROWS - N)])
    return (src.reshape(NT, NBI, BK, CH), dst.reshape(NT, NBI, BK, CH))


def kernel(x1, edge_index1, x2, edge_index2, W1, b1, W2, b2, W3, b3,
           Wa1, Wa2, Wa3, T1, Tb1, Tc1, T2, Tb2, Tc2, T3, Tb3, Tc3,
           Ws1, bs1, Ws2, bs2):
    x_st = jnp.concatenate([x1, x2], axis=0)
    s1, d1 = _pad_edges(edge_index1, 0)
    s2, d2 = _pad_edges(edge_index2, 1)
    srcs = jnp.stack([s1, s2])
    dsts = jnp.stack([d1, d2])
    b1r, b2r, b3r = b1[None, :], b2[None, :], b3[None, :]
    t1f = T1.reshape(128, 128 * 128)
    t2f = T2.reshape(64, 64 * 64)
    t3f = T3.reshape(32, 32 * 32)
    tb1t, tb2t, tb3t = Tb1.T, Tb2.T, Tb3.T
    tc1r, tc2r, tc3r = Tc1.T, Tc2.T, Tc3.T
    bs1r, bs2r = bs1[None, :], bs2[None, :]

    h1_raw = _tc_k1a(x_st, W1)
    deg_st = _deg_kernel(dsts)

    h1 = _tc_k1b(h1_raw, deg_st)
    a1 = _agg_kernel(128)(h1, srcs, dsts)
    f1, h2, cs1 = _tc_layer(a1, deg_st, b1r, W2, 128, 64)
    a2 = _agg_kernel(64)(h2, srcs, dsts)
    p1 = _tc_attn1(f1, cs1, Wa1, 128)
    f2, h3, cs2 = _tc_layer(a2, deg_st, b2r, W3, 64, 32)
    a3 = _agg_kernel(32)(h3, srcs, dsts)
    p2 = _tc_attn1(f2, cs2, Wa2, 64)
    f3, cs3 = _tc_last(a3, deg_st, b3r, 32)
    p3 = _tc_attn1(f3, cs3, Wa3, 32)

    o1, o2, o3 = _tc_ntn_a(p1, p2, p3, t1f, t2f, t3f)
    m1 = o1.reshape(128, 128)
    m2 = o2.reshape(64, 64)
    m3 = o3.reshape(32, 32)
    return _tc_ntn_b(m1, m2, m3, p1, p2, p3, tb1t, tb2t, tb3t,
                     tc1r, tc2r, tc3r, Ws1, bs1r, Ws2, bs2r)


# speedup vs baseline: 1.1591x; 1.1591x over previous
import functools

import jax
import jax.numpy as jnp
from jax import lax
from jax.experimental import pallas as pl
from jax.experimental.pallas import tpu as pltpu
from jax.experimental.pallas import tpu_sc as plsc

N = 10000
E = 320000
NT = 16
NC = 2
EPT = 20480
CHUNK128 = (64, 32, 10)
CHUNK_W = (128, 8, 20)
EPAD = NT * EPT
AGG_ROWS = 10048
DEG_ROWS = 10240
RPT = 640
RPT_LAST = N - (NT - 1) * RPT
R = 2000
NBLK = (2 * N) // R

@functools.cache
def _get_mesh():
    return plsc.VectorSubcoreMesh(core_axis_name="c", subcore_axis_name="s",
                                  num_cores=NC, num_subcores=NT)


def _deg_body(dsts_hbm, out_hbm, ones_v, dst_v, acc_sh):
    CH, BK, NBI = CHUNK_W
    cid = lax.axis_index("c")
    sid = lax.axis_index("s")

    def _fill(i, carry):
        ones_v[i] = jnp.ones((16,), jnp.float32)
        return carry

    lax.fori_loop(0, CH, _fill, 0)
    for k in range(DEG_ROWS // NT // CH):
        pltpu.sync_copy(ones_v, acc_sh.at[pl.ds(sid * (DEG_ROWS // NT) + k * CH, CH)])
    pltpu.sync_copy(dsts_hbm.at[cid, sid], dst_v)
    plsc.subcore_barrier()

    def _blk(b, carry):
        for k in range(BK):
            pltpu.sync_copy(ones_v, acc_sh.at[dst_v.at[b, k]], add=True)
        return carry

    lax.fori_loop(0, NBI, _blk, 0)
    plsc.subcore_barrier()

    @pl.when(sid < NT - 1)
    def _():
        pltpu.sync_copy(acc_sh.at[pl.ds(sid * RPT, RPT)],
                        out_hbm.at[pl.ds(cid * N + sid * RPT, RPT)])

    @pl.when(sid == NT - 1)
    def _():
        pltpu.sync_copy(acc_sh.at[pl.ds((NT - 1) * RPT, RPT_LAST)],
                        out_hbm.at[pl.ds(cid * N + (NT - 1) * RPT, RPT_LAST)])


@functools.cache
def _deg_kernel_fn():
    return pl.kernel(
        _deg_body,
        out_type=jax.ShapeDtypeStruct((2 * N, 16), jnp.float32),
        mesh=_get_mesh(),
        scratch_types=[
            pltpu.VMEM((CHUNK_W[0], 16), jnp.float32),
            pltpu.VMEM((CHUNK_W[2], CHUNK_W[1], CHUNK_W[0]), jnp.int32),
            pltpu.VMEM_SHARED((DEG_ROWS, 16), jnp.float32),
        ],
        compiler_params=pltpu.CompilerParams(use_tc_tiling_on_sc=False),
    )


def _deg_kernel(dsts):
    return _deg_kernel_fn()(dsts)


def _make_agg_body(CH, BK, NBI):
  CPT = CH and (EPT // CH)
  def _agg_body(h_hbm, srcs_hbm, dsts_hbm, out_hbm,
              sidx, didx, rows_v, acc_sh, sem_i, sem_g, sem_s):
      cid = lax.axis_index("c")
      sid = lax.axis_index("s")
      @pl.when(sid < NT - 1)
      def _():
          pltpu.sync_copy(h_hbm.at[pl.ds(cid * N + sid * RPT, RPT)],
                          acc_sh.at[pl.ds(sid * RPT, RPT)])

      @pl.when(sid == NT - 1)
      def _():
          pltpu.sync_copy(h_hbm.at[pl.ds(cid * N + (NT - 1) * RPT, RPT_LAST)],
                          acc_sh.at[pl.ds((NT - 1) * RPT, RPT_LAST)])

      pltpu.sync_copy(srcs_hbm.at[cid, sid, 0], sidx.at[0])
      pltpu.sync_copy(dsts_hbm.at[cid, sid, 0], didx.at[0])
      plsc.subcore_barrier()
      pltpu.async_copy(h_hbm.at[sidx.at[0, 0]], rows_v.at[0], sem_g.at[0])
      pltpu.async_copy(h_hbm.at[sidx.at[0, 1]], rows_v.at[1], sem_g.at[1])

      def _blk(b, carry):
          islot = lax.rem(b, 2)

          @pl.when(b + 1 < NBI)
          def _():
              pltpu.async_copy(srcs_hbm.at[cid, sid, b + 1],
                               sidx.at[1 - islot], sem_i)
              pltpu.async_copy(dsts_hbm.at[cid, sid, b + 1],
                               didx.at[1 - islot], sem_i)

          def _chunk(k, carry2):
              c = b * BK + k

              @pl.when(jnp.logical_and(k == BK - 2, b + 1 < NBI))
              def _():
                  pltpu.make_async_copy(srcs_hbm.at[cid, sid, b + 1],
                                        sidx.at[1 - islot], sem_i).wait()
                  pltpu.make_async_copy(dsts_hbm.at[cid, sid, b + 1],
                                        didx.at[1 - islot], sem_i).wait()

              c2 = c + 2
              slot = lax.rem(c, 3)
              slot2 = lax.rem(c2, 3)
              is2 = lax.rem(c2 // BK, 2)
              row2 = lax.rem(c2, BK)

              @pl.when(c2 < CPT)
              def _():
                  @pl.when(c >= 1)
                  def _():
                      pltpu.make_async_copy(
                          rows_v.at[slot2], acc_sh.at[didx.at[is2, row2]],
                          sem_s.at[slot2]).wait()

                  pltpu.async_copy(h_hbm.at[sidx.at[is2, row2]],
                                   rows_v.at[slot2], sem_g.at[slot2])

              pltpu.make_async_copy(h_hbm.at[sidx.at[islot, k]],
                                    rows_v.at[slot], sem_g.at[slot]).wait()
              pltpu.async_copy(rows_v.at[slot], acc_sh.at[didx.at[islot, k]],
                               sem_s.at[slot], add=True)
              return carry2

          lax.fori_loop(0, BK, _chunk, 0)
          return carry

      lax.fori_loop(0, NBI, _blk, 0)
      for c in (CPT - 3, CPT - 2, CPT - 1):
          slot = c % 3
          pltpu.make_async_copy(rows_v.at[slot],
                                acc_sh.at[didx.at[(c // BK) % 2, c % BK]],
                                sem_s.at[slot]).wait()
      plsc.subcore_barrier()

      @pl.when(sid < NT - 1)
      def _():
          pltpu.sync_copy(acc_sh.at[pl.ds(sid * RPT, RPT)],
                          out_hbm.at[pl.ds(cid * N + sid * RPT, RPT)])

      @pl.when(sid == NT - 1)
      def _():
          pltpu.sync_copy(acc_sh.at[pl.ds((NT - 1) * RPT, RPT_LAST)],
                          out_hbm.at[pl.ds(cid * N + (NT - 1) * RPT, RPT_LAST)])

  return _agg_body


@functools.cache
def _agg_kernel_fn(F):
    CH, BK, NBI = CHUNK128 if F == 128 else CHUNK_W
    return pl.kernel(
        _make_agg_body(CH, BK, NBI),
        out_type=jax.ShapeDtypeStruct((2 * N, F), jnp.float32),
        mesh=_get_mesh(),
        scratch_types=[
            pltpu.VMEM((2, BK, CH), jnp.int32),
            pltpu.VMEM((2, BK, CH), jnp.int32),
            pltpu.VMEM((3, CH, F), jnp.float32),
            pltpu.VMEM_SHARED((AGG_ROWS, F), jnp.float32),
            pltpu.SemaphoreType.DMA,
            pltpu.SemaphoreType.DMA((3,)),
            pltpu.SemaphoreType.DMA((3,)),
        ],
        compiler_params=pltpu.CompilerParams(use_tc_tiling_on_sc=False)
        if F < 128 else None,
    )


def _agg_kernel(F):
    return _agg_kernel_fn(F)


def _k1a_body(x_ref, w_ref, out_ref):
    out_ref[...] = jnp.dot(x_ref[...], w_ref[...],
                           preferred_element_type=jnp.float32)


def _k1b_body(h_ref, deg_ref, out_ref):
    dinv = lax.rsqrt(deg_ref[...][:, :1])
    out_ref[...] = h_ref[...] * dinv


def _layer_body(acc_ref, deg_ref, b_ref, w_ref, f_ref, h_ref, cs_ref):
    i = pl.program_id(0)
    dinv = lax.rsqrt(deg_ref[...][:, :1])
    f = acc_ref[...] * dinv + b_ref[...]
    f_ref[...] = f
    r = jnp.maximum(f, 0.0)
    h_ref[...] = jnp.dot(r, w_ref[...], preferred_element_type=jnp.float32) * dinv

    @pl.when(i % (NBLK // 2) == 0)
    def _():
        cs_ref[...] = jnp.zeros_like(cs_ref)

    cs_ref[...] += jnp.sum(f, axis=0)[None, None, :]


def _last_body(acc_ref, deg_ref, b_ref, f_ref, cs_ref):
    i = pl.program_id(0)
    dinv = lax.rsqrt(deg_ref[...][:, :1])
    f = acc_ref[...] * dinv + b_ref[...]
    f_ref[...] = f

    @pl.when(i % (NBLK // 2) == 0)
    def _():
        cs_ref[...] = jnp.zeros_like(cs_ref)

    cs_ref[...] += jnp.sum(f, axis=0)[None, None, :]


def _attn_body(f_ref, cs_ref, wa_ref, p_ref):
    i = pl.program_id(0)
    gc = jnp.tanh(jnp.dot(cs_ref[0] * (1.0 / N), wa_ref[...],
                          preferred_element_type=jnp.float32))
    f = f_ref[...]
    s = jax.nn.sigmoid(
        lax.dot_general(f, gc, (((1,), (1,)), ((), ())),
                        preferred_element_type=jnp.float32))
    contrib = lax.dot_general(s, f, (((0,), (0,)), ((), ())),
                              preferred_element_type=jnp.float32)

    @pl.when(i % (NBLK // 2) == 0)
    def _():
        p_ref[...] = jnp.zeros_like(p_ref)

    p_ref[...] += contrib[None]


def _ntn_a_body(p1_ref, p2_ref, p3_ref, t1_ref, t2_ref, t3_ref,
                o1_ref, o2_ref, o3_ref):
    for p_ref, t_ref, o_ref in ((p1_ref, t1_ref, o1_ref),
                                (p2_ref, t2_ref, o2_ref),
                                (p3_ref, t3_ref, o3_ref)):
        o_ref[...] = jnp.dot(p_ref[0], t_ref[...],
                             preferred_element_type=jnp.float32)


def _ntn_b_body(m1_ref, m2_ref, m3_ref, p1_ref, p2_ref, p3_ref,
                tb1_ref, tb2_ref, tb3_ref, tc1_ref, tc2_ref, tc3_ref,
                ws1_ref, bs1_ref, ws2_ref, bs2_ref, out_ref):
    parts = []
    for p_ref, m_ref, tbt_ref, tcr_ref in (
            (p1_ref, m1_ref, tb1_ref, tc1_ref),
            (p2_ref, m2_ref, tb2_ref, tc2_ref),
            (p3_ref, m3_ref, tb3_ref, tc3_ref)):
        e1 = p_ref[0]
        e2 = p_ref[1]
        scoring = jnp.dot(e2, m_ref[...], preferred_element_type=jnp.float32)
        comb = jnp.concatenate([e1, e2], axis=1)
        blk = jnp.dot(comb, tbt_ref[...], preferred_element_type=jnp.float32)
        parts.append(jnp.maximum(scoring + blk + tcr_ref[...], 0.0))
    scores = jnp.concatenate(parts, axis=1)
    h = jnp.maximum(jnp.dot(scores, ws1_ref[...],
                            preferred_element_type=jnp.float32) + bs1_ref[...], 0.0)
    out_ref[...] = jax.nn.sigmoid(
        jnp.dot(h, ws2_ref[...], preferred_element_type=jnp.float32) + bs2_ref[...])


def _row_spec(F):
    return pl.BlockSpec((R, F), lambda i: (i, 0))


def _full_spec(shape):
    nd = len(shape)
    return pl.BlockSpec(shape, lambda i, _n=nd: (0,) * _n)


def _cs_spec(F):
    return pl.BlockSpec((1, 1, F), lambda i: (i // (NBLK // 2), 0, 0))


def _tc_k1a(x_st, w1):
    return pl.pallas_call(
        _k1a_body,
        grid=(NBLK,),
        in_specs=[_row_spec(128), _full_spec((128, 128))],
        out_specs=_row_spec(128),
        out_shape=jax.ShapeDtypeStruct((2 * N, 128), jnp.float32),
    )(x_st, w1)


def _tc_k1b(h_raw, deg_st):
    return pl.pallas_call(
        _k1b_body,
        grid=(NBLK,),
        in_specs=[_row_spec(128), _row_spec(16)],
        out_specs=_row_spec(128),
        out_shape=jax.ShapeDtypeStruct((2 * N, 128), jnp.float32),
    )(h_raw, deg_st)


def _tc_layer(acc_st, deg_st, b_row, w_next, Fi, Fo):
    return pl.pallas_call(
        _layer_body,
        grid=(NBLK,),
        in_specs=[_row_spec(Fi), _row_spec(16), _full_spec((1, Fi)),
                  _full_spec((Fi, Fo))],
        out_specs=[_row_spec(Fi), _row_spec(Fo), _cs_spec(Fi)],
        out_shape=[jax.ShapeDtypeStruct((2 * N, Fi), jnp.float32),
                   jax.ShapeDtypeStruct((2 * N, Fo), jnp.float32),
                   jax.ShapeDtypeStruct((NC, 1, Fi), jnp.float32)],
    )(acc_st, deg_st, b_row, w_next)


def _tc_last(acc_st, deg_st, b_row, Fi):
    return pl.pallas_call(
        _last_body,
        grid=(NBLK,),
        in_specs=[_row_spec(Fi), _row_spec(16), _full_spec((1, Fi))],
        out_specs=[_row_spec(Fi), _cs_spec(Fi)],
        out_shape=[jax.ShapeDtypeStruct((2 * N, Fi), jnp.float32),
                   jax.ShapeDtypeStruct((NC, 1, Fi), jnp.float32)],
    )(acc_st, deg_st, b_row)


def _tc_attn1(f, cs, wa, F):
    return pl.pallas_call(
        _attn_body,
        grid=(NBLK,),
        in_specs=[_row_spec(F), _cs_spec(F), _full_spec((F, F))],
        out_specs=_cs_spec(F),
        out_shape=jax.ShapeDtypeStruct((NC, 1, F), jnp.float32),
    )(f, cs, wa)


def _tc_ntn_a(p1, p2, p3, t1f, t2f, t3f):
    return pl.pallas_call(
        _ntn_a_body,
        grid=(1,),
        in_specs=[_full_spec((NC, 1, 128)), _full_spec((NC, 1, 64)),
                  _full_spec((NC, 1, 32)),
                  _full_spec((128, 128 * 128)), _full_spec((64, 64 * 64)),
                  _full_spec((32, 32 * 32))],
        out_specs=[_full_spec((1, 128 * 128)), _full_spec((1, 64 * 64)),
                   _full_spec((1, 32 * 32))],
        out_shape=[jax.ShapeDtypeStruct((1, 128 * 128), jnp.float32),
                   jax.ShapeDtypeStruct((1, 64 * 64), jnp.float32),
                   jax.ShapeDtypeStruct((1, 32 * 32), jnp.float32)],
    )(p1, p2, p3, t1f, t2f, t3f)


def _tc_ntn_b(m1, m2, m3, p1, p2, p3, tb1t, tb2t, tb3t, tc1r, tc2r, tc3r,
              ws1, bs1r, ws2, bs2r):
    return pl.pallas_call(
        _ntn_b_body,
        grid=(1,),
        in_specs=[_full_spec((128, 128)), _full_spec((64, 64)),
                  _full_spec((32, 32)),
                  _full_spec((NC, 1, 128)), _full_spec((NC, 1, 64)),
                  _full_spec((NC, 1, 32)),
                  _full_spec((256, 128)), _full_spec((128, 64)),
                  _full_spec((64, 32)),
                  _full_spec((1, 128)), _full_spec((1, 64)), _full_spec((1, 32)),
                  _full_spec((224, 16)), _full_spec((1, 16)),
                  _full_spec((16, 1)), _full_spec((1, 1))],
        out_specs=_full_spec((1, 1)),
        out_shape=jax.ShapeDtypeStruct((1, 1), jnp.float32),
    )(m1, m2, m3, p1, p2, p3, tb1t, tb2t, tb3t, tc1r, tc2r, tc3r,
      ws1, bs1r, ws2, bs2r)


def _pad_edges(ei, g):
    npad = EPAD - E
    ar = jnp.arange(npad, dtype=jnp.int32)
    flat = ei.reshape(2 * E)
    src = jnp.concatenate([flat[:E] + g * N, ar % 256 + g * N])
    dst = jnp.concatenate([flat[E:], N + ar % (AGG_ROWS - N)])
    return src, dst


def kernel(x1, edge_index1, x2, edge_index2, W1, b1, W2, b2, W3, b3,
           Wa1, Wa2, Wa3, T1, Tb1, Tc1, T2, Tb2, Tc2, T3, Tb3, Tc3,
           Ws1, bs1, Ws2, bs2):
    x_st = jnp.concatenate([x1, x2], axis=0)
    s1, d1 = _pad_edges(edge_index1, 0)
    s2, d2 = _pad_edges(edge_index2, 1)
    srcs = jnp.stack([s1, s2])
    dsts = jnp.stack([d1, d2])

    def _chunked(a, chk):
        ch, bk, nbi = chk
        return a.reshape(NC, NT, nbi, bk, ch)

    srcs_n, dsts_n = _chunked(srcs, CHUNK128), _chunked(dsts, CHUNK128)
    srcs_w, dsts_w = _chunked(srcs, CHUNK_W), _chunked(dsts, CHUNK_W)
    b1r, b2r, b3r = b1[None, :], b2[None, :], b3[None, :]
    t1f = T1.reshape(128, 128 * 128)
    t2f = T2.reshape(64, 64 * 64)
    t3f = T3.reshape(32, 32 * 32)
    tb1t, tb2t, tb3t = Tb1.T, Tb2.T, Tb3.T
    tc1r, tc2r, tc3r = Tc1.T, Tc2.T, Tc3.T
    bs1r, bs2r = bs1[None, :], bs2[None, :]

    h1_raw = _tc_k1a(x_st, W1)
    deg_st = _deg_kernel(dsts_w)

    h1 = _tc_k1b(h1_raw, deg_st)
    a1 = _agg_kernel(128)(h1, srcs_n, dsts_n)
    f1, h2, cs1 = _tc_layer(a1, deg_st, b1r, W2, 128, 64)
    a2 = _agg_kernel(64)(h2, srcs_w, dsts_w)
    p1 = _tc_attn1(f1, cs1, Wa1, 128)
    f2, h3, cs2 = _tc_layer(a2, deg_st, b2r, W3, 64, 32)
    a3 = _agg_kernel(32)(h3, srcs_w, dsts_w)
    p2 = _tc_attn1(f2, cs2, Wa2, 64)
    f3, cs3 = _tc_last(a3, deg_st, b3r, 32)
    p3 = _tc_attn1(f3, cs3, Wa3, 32)

    o1, o2, o3 = _tc_ntn_a(p1, p2, p3, t1f, t2f, t3f)
    m1 = o1.reshape(128, 128)
    m2 = o2.reshape(64, 64)
    m3 = o3.reshape(32, 32)
    return _tc_ntn_b(m1, m2, m3, p1, p2, p3, tb1t, tb2t, tb3t,
                     tc1r, tc2r, tc3r, Ws1, bs1r, Ws2, bs2r)


# speedup vs baseline: 1.4157x; 1.2214x over previous
import functools

import jax
import jax.numpy as jnp
from jax import lax
from jax.experimental import pallas as pl
from jax.experimental.pallas import tpu as pltpu
from jax.experimental.pallas import tpu_sc as plsc

N = 10000
E = 320000
NT = 16
NC = 2
EPT = 20480
CHUNK_W = (128, 8, 20)
EPAD = NT * EPT
AGG_ROWS = 10048
DEG_ROWS = 10240
RPT = 640
RPT_LAST = N - (NT - 1) * RPT
R = 2000
NBLK = (2 * N) // R

@functools.cache
def _get_mesh():
    return plsc.VectorSubcoreMesh(core_axis_name="c", subcore_axis_name="s",
                                  num_cores=NC, num_subcores=NT)


def _deg_body(dsts_hbm, out_hbm, ones_v, dst_v, acc_sh):
    CH, BK, NBI = CHUNK_W
    cid = lax.axis_index("c")
    sid = lax.axis_index("s")

    def _fill(i, carry):
        ones_v[i] = jnp.ones((16,), jnp.float32)
        return carry

    lax.fori_loop(0, CH, _fill, 0)
    for k in range(DEG_ROWS // NT // CH):
        pltpu.sync_copy(ones_v, acc_sh.at[pl.ds(sid * (DEG_ROWS // NT) + k * CH, CH)])
    pltpu.sync_copy(dsts_hbm.at[cid, sid], dst_v)
    plsc.subcore_barrier()

    def _blk(b, carry):
        for k in range(BK):
            pltpu.sync_copy(ones_v, acc_sh.at[dst_v.at[b, k]], add=True)
        return carry

    lax.fori_loop(0, NBI, _blk, 0)
    plsc.subcore_barrier()

    @pl.when(sid < NT - 1)
    def _():
        pltpu.sync_copy(acc_sh.at[pl.ds(sid * RPT, RPT)],
                        out_hbm.at[pl.ds(cid * N + sid * RPT, RPT)])

    @pl.when(sid == NT - 1)
    def _():
        pltpu.sync_copy(acc_sh.at[pl.ds((NT - 1) * RPT, RPT_LAST)],
                        out_hbm.at[pl.ds(cid * N + (NT - 1) * RPT, RPT_LAST)])


@functools.cache
def _deg_kernel_fn():
    return pl.kernel(
        _deg_body,
        out_type=jax.ShapeDtypeStruct((2 * N, 16), jnp.float32),
        mesh=_get_mesh(),
        scratch_types=[
            pltpu.VMEM((CHUNK_W[0], 16), jnp.float32),
            pltpu.VMEM((CHUNK_W[2], CHUNK_W[1], CHUNK_W[0]), jnp.int32),
            pltpu.VMEM_SHARED((DEG_ROWS, 16), jnp.float32),
        ],
        compiler_params=pltpu.CompilerParams(use_tc_tiling_on_sc=False),
    )


def _deg_kernel(dsts):
    return _deg_kernel_fn()(dsts)


def _make_agg_body(CH, BK, NBI):
  CPT = CH and (EPT // CH)
  def _agg_body(h_hbm, srcs_hbm, dsts_hbm, out_hbm,
              sidx, didx, rows_v, acc_sh, sem_i, sem_g, sem_s):
      cid = lax.axis_index("c")
      sid = lax.axis_index("s")
      @pl.when(sid < NT - 1)
      def _():
          pltpu.sync_copy(h_hbm.at[pl.ds(cid * N + sid * RPT, RPT)],
                          acc_sh.at[pl.ds(sid * RPT, RPT)])

      @pl.when(sid == NT - 1)
      def _():
          pltpu.sync_copy(h_hbm.at[pl.ds(cid * N + (NT - 1) * RPT, RPT_LAST)],
                          acc_sh.at[pl.ds((NT - 1) * RPT, RPT_LAST)])

      pltpu.sync_copy(srcs_hbm.at[cid, sid, 0], sidx.at[0])
      pltpu.sync_copy(dsts_hbm.at[cid, sid, 0], didx.at[0])
      plsc.subcore_barrier()
      pltpu.async_copy(h_hbm.at[sidx.at[0, 0]], rows_v.at[0], sem_g.at[0])
      pltpu.async_copy(h_hbm.at[sidx.at[0, 1]], rows_v.at[1], sem_g.at[1])

      def _blk(b, carry):
          islot = lax.rem(b, 2)

          @pl.when(b + 1 < NBI)
          def _():
              pltpu.async_copy(srcs_hbm.at[cid, sid, b + 1],
                               sidx.at[1 - islot], sem_i)
              pltpu.async_copy(dsts_hbm.at[cid, sid, b + 1],
                               didx.at[1 - islot], sem_i)

          def _chunk(k, carry2):
              c = b * BK + k

              @pl.when(jnp.logical_and(k == BK - 2, b + 1 < NBI))
              def _():
                  pltpu.make_async_copy(srcs_hbm.at[cid, sid, b + 1],
                                        sidx.at[1 - islot], sem_i).wait()
                  pltpu.make_async_copy(dsts_hbm.at[cid, sid, b + 1],
                                        didx.at[1 - islot], sem_i).wait()

              c2 = c + 2
              slot = lax.rem(c, 3)
              slot2 = lax.rem(c2, 3)
              is2 = lax.rem(c2 // BK, 2)
              row2 = lax.rem(c2, BK)

              @pl.when(c2 < CPT)
              def _():
                  @pl.when(c >= 1)
                  def _():
                      pltpu.make_async_copy(
                          rows_v.at[slot2], acc_sh.at[didx.at[is2, row2]],
                          sem_s.at[slot2]).wait()

                  pltpu.async_copy(h_hbm.at[sidx.at[is2, row2]],
                                   rows_v.at[slot2], sem_g.at[slot2])

              pltpu.make_async_copy(h_hbm.at[sidx.at[islot, k]],
                                    rows_v.at[slot], sem_g.at[slot]).wait()
              pltpu.async_copy(rows_v.at[slot], acc_sh.at[didx.at[islot, k]],
                               sem_s.at[slot], add=True)
              return carry2

          lax.fori_loop(0, BK, _chunk, 0)
          return carry

      lax.fori_loop(0, NBI, _blk, 0)
      for c in (CPT - 3, CPT - 2, CPT - 1):
          slot = c % 3
          pltpu.make_async_copy(rows_v.at[slot],
                                acc_sh.at[didx.at[(c // BK) % 2, c % BK]],
                                sem_s.at[slot]).wait()
      plsc.subcore_barrier()

      @pl.when(sid < NT - 1)
      def _():
          pltpu.sync_copy(acc_sh.at[pl.ds(sid * RPT, RPT)],
                          out_hbm.at[pl.ds(cid * N + sid * RPT, RPT)])

      @pl.when(sid == NT - 1)
      def _():
          pltpu.sync_copy(acc_sh.at[pl.ds((NT - 1) * RPT, RPT_LAST)],
                          out_hbm.at[pl.ds(cid * N + (NT - 1) * RPT, RPT_LAST)])

  return _agg_body


@functools.cache
def _agg_kernel_fn(F):
    CH, BK, NBI = CHUNK_W
    return pl.kernel(
        _make_agg_body(CH, BK, NBI),
        out_type=jax.ShapeDtypeStruct((2 * N, F), jnp.bfloat16),
        mesh=_get_mesh(),
        scratch_types=[
            pltpu.VMEM((2, BK, CH), jnp.int32),
            pltpu.VMEM((2, BK, CH), jnp.int32),
            pltpu.VMEM((3, CH, F), jnp.bfloat16),
            pltpu.VMEM_SHARED((AGG_ROWS, F), jnp.bfloat16),
            pltpu.SemaphoreType.DMA,
            pltpu.SemaphoreType.DMA((3,)),
            pltpu.SemaphoreType.DMA((3,)),
        ],
        compiler_params=pltpu.CompilerParams(use_tc_tiling_on_sc=False),
    )


def _agg_kernel(F):
    return _agg_kernel_fn(F)


def _k1a_body(x_ref, w_ref, out_ref):
    out_ref[...] = jnp.dot(x_ref[...], w_ref[...],
                           preferred_element_type=jnp.float32)


def _k1b_body(h_ref, deg_ref, out_ref):
    dinv = lax.rsqrt(deg_ref[...][:, :1])
    out_ref[...] = (h_ref[...] * dinv).astype(jnp.bfloat16)


def _layer_body(acc_ref, deg_ref, b_ref, w_ref, f_ref, h_ref, cs_ref):
    i = pl.program_id(0)
    dinv = lax.rsqrt(deg_ref[...][:, :1])
    f = acc_ref[...].astype(jnp.float32) * dinv + b_ref[...]
    f_ref[...] = f
    r = jnp.maximum(f, 0.0)
    h_ref[...] = (jnp.dot(r, w_ref[...], preferred_element_type=jnp.float32)
                  * dinv).astype(jnp.bfloat16)

    @pl.when(i % (NBLK // 2) == 0)
    def _():
        cs_ref[...] = jnp.zeros_like(cs_ref)

    cs_ref[...] += jnp.sum(f, axis=0)[None, None, :]


def _last_body(acc_ref, deg_ref, b_ref, f_ref, cs_ref):
    i = pl.program_id(0)
    dinv = lax.rsqrt(deg_ref[...][:, :1])
    f = acc_ref[...].astype(jnp.float32) * dinv + b_ref[...]
    f_ref[...] = f

    @pl.when(i % (NBLK // 2) == 0)
    def _():
        cs_ref[...] = jnp.zeros_like(cs_ref)

    cs_ref[...] += jnp.sum(f, axis=0)[None, None, :]


def _attn_body(f_ref, cs_ref, wa_ref, p_ref):
    i = pl.program_id(0)
    gc = jnp.tanh(jnp.dot(cs_ref[0] * (1.0 / N), wa_ref[...],
                          preferred_element_type=jnp.float32))
    f = f_ref[...]
    s = jax.nn.sigmoid(
        lax.dot_general(f, gc, (((1,), (1,)), ((), ())),
                        preferred_element_type=jnp.float32))
    contrib = lax.dot_general(s, f, (((0,), (0,)), ((), ())),
                              preferred_element_type=jnp.float32)

    @pl.when(i % (NBLK // 2) == 0)
    def _():
        p_ref[...] = jnp.zeros_like(p_ref)

    p_ref[...] += contrib[None]


def _ntn_a_body(p1_ref, p2_ref, p3_ref, t1_ref, t2_ref, t3_ref,
                o1_ref, o2_ref, o3_ref):
    for p_ref, t_ref, o_ref in ((p1_ref, t1_ref, o1_ref),
                                (p2_ref, t2_ref, o2_ref),
                                (p3_ref, t3_ref, o3_ref)):
        o_ref[...] = jnp.dot(p_ref[0], t_ref[...],
                             preferred_element_type=jnp.float32)


def _ntn_b_body(m1_ref, m2_ref, m3_ref, p1_ref, p2_ref, p3_ref,
                tb1_ref, tb2_ref, tb3_ref, tc1_ref, tc2_ref, tc3_ref,
                ws1_ref, bs1_ref, ws2_ref, bs2_ref, out_ref):
    parts = []
    for p_ref, m_ref, tbt_ref, tcr_ref in (
            (p1_ref, m1_ref, tb1_ref, tc1_ref),
            (p2_ref, m2_ref, tb2_ref, tc2_ref),
            (p3_ref, m3_ref, tb3_ref, tc3_ref)):
        e1 = p_ref[0]
        e2 = p_ref[1]
        scoring = jnp.dot(e2, m_ref[...], preferred_element_type=jnp.float32)
        comb = jnp.concatenate([e1, e2], axis=1)
        blk = jnp.dot(comb, tbt_ref[...], preferred_element_type=jnp.float32)
        parts.append(jnp.maximum(scoring + blk + tcr_ref[...], 0.0))
    scores = jnp.concatenate(parts, axis=1)
    h = jnp.maximum(jnp.dot(scores, ws1_ref[...],
                            preferred_element_type=jnp.float32) + bs1_ref[...], 0.0)
    out_ref[...] = jax.nn.sigmoid(
        jnp.dot(h, ws2_ref[...], preferred_element_type=jnp.float32) + bs2_ref[...])


def _row_spec(F):
    return pl.BlockSpec((R, F), lambda i: (i, 0))


def _full_spec(shape):
    nd = len(shape)
    return pl.BlockSpec(shape, lambda i, _n=nd: (0,) * _n)


def _cs_spec(F):
    return pl.BlockSpec((1, 1, F), lambda i: (i // (NBLK // 2), 0, 0))


def _tc_k1a(x_st, w1):
    return pl.pallas_call(
        _k1a_body,
        grid=(NBLK,),
        in_specs=[_row_spec(128), _full_spec((128, 128))],
        out_specs=_row_spec(128),
        out_shape=jax.ShapeDtypeStruct((2 * N, 128), jnp.float32),
    )(x_st, w1)


def _tc_k1b(h_raw, deg_st):
    return pl.pallas_call(
        _k1b_body,
        grid=(NBLK,),
        in_specs=[_row_spec(128), _row_spec(16)],
        out_specs=_row_spec(128),
        out_shape=jax.ShapeDtypeStruct((2 * N, 128), jnp.bfloat16),
    )(h_raw, deg_st)


def _tc_layer(acc_st, deg_st, b_row, w_next, Fi, Fo):
    return pl.pallas_call(
        _layer_body,
        grid=(NBLK,),
        in_specs=[_row_spec(Fi), _row_spec(16), _full_spec((1, Fi)),
                  _full_spec((Fi, Fo))],
        out_specs=[_row_spec(Fi), _row_spec(Fo), _cs_spec(Fi)],
        out_shape=[jax.ShapeDtypeStruct((2 * N, Fi), jnp.float32),
                   jax.ShapeDtypeStruct((2 * N, Fo), jnp.bfloat16),
                   jax.ShapeDtypeStruct((NC, 1, Fi), jnp.float32)],
    )(acc_st, deg_st, b_row, w_next)


def _tc_last(acc_st, deg_st, b_row, Fi):
    return pl.pallas_call(
        _last_body,
        grid=(NBLK,),
        in_specs=[_row_spec(Fi), _row_spec(16), _full_spec((1, Fi))],
        out_specs=[_row_spec(Fi), _cs_spec(Fi)],
        out_shape=[jax.ShapeDtypeStruct((2 * N, Fi), jnp.float32),
                   jax.ShapeDtypeStruct((NC, 1, Fi), jnp.float32)],
    )(acc_st, deg_st, b_row)


def _tc_attn1(f, cs, wa, F):
    return pl.pallas_call(
        _attn_body,
        grid=(NBLK,),
        in_specs=[_row_spec(F), _cs_spec(F), _full_spec((F, F))],
        out_specs=_cs_spec(F),
        out_shape=jax.ShapeDtypeStruct((NC, 1, F), jnp.float32),
    )(f, cs, wa)


def _tc_ntn_a(p1, p2, p3, t1f, t2f, t3f):
    return pl.pallas_call(
        _ntn_a_body,
        grid=(1,),
        in_specs=[_full_spec((NC, 1, 128)), _full_spec((NC, 1, 64)),
                  _full_spec((NC, 1, 32)),
                  _full_spec((128, 128 * 128)), _full_spec((64, 64 * 64)),
                  _full_spec((32, 32 * 32))],
        out_specs=[_full_spec((1, 128 * 128)), _full_spec((1, 64 * 64)),
                   _full_spec((1, 32 * 32))],
        out_shape=[jax.ShapeDtypeStruct((1, 128 * 128), jnp.float32),
                   jax.ShapeDtypeStruct((1, 64 * 64), jnp.float32),
                   jax.ShapeDtypeStruct((1, 32 * 32), jnp.float32)],
    )(p1, p2, p3, t1f, t2f, t3f)


def _tc_ntn_b(m1, m2, m3, p1, p2, p3, tb1t, tb2t, tb3t, tc1r, tc2r, tc3r,
              ws1, bs1r, ws2, bs2r):
    return pl.pallas_call(
        _ntn_b_body,
        grid=(1,),
        in_specs=[_full_spec((128, 128)), _full_spec((64, 64)),
                  _full_spec((32, 32)),
                  _full_spec((NC, 1, 128)), _full_spec((NC, 1, 64)),
                  _full_spec((NC, 1, 32)),
                  _full_spec((256, 128)), _full_spec((128, 64)),
                  _full_spec((64, 32)),
                  _full_spec((1, 128)), _full_spec((1, 64)), _full_spec((1, 32)),
                  _full_spec((224, 16)), _full_spec((1, 16)),
                  _full_spec((16, 1)), _full_spec((1, 1))],
        out_specs=_full_spec((1, 1)),
        out_shape=jax.ShapeDtypeStruct((1, 1), jnp.float32),
    )(m1, m2, m3, p1, p2, p3, tb1t, tb2t, tb3t, tc1r, tc2r, tc3r,
      ws1, bs1r, ws2, bs2r)


def _pad_edges(ei, g):
    npad = EPAD - E
    ar = jnp.arange(npad, dtype=jnp.int32)
    flat = ei.reshape(2 * E)
    src = jnp.concatenate([flat[:E] + g * N, ar % 256 + g * N])
    dst = jnp.concatenate([flat[E:], N + ar % (AGG_ROWS - N)])
    return src, dst


def kernel(x1, edge_index1, x2, edge_index2, W1, b1, W2, b2, W3, b3,
           Wa1, Wa2, Wa3, T1, Tb1, Tc1, T2, Tb2, Tc2, T3, Tb3, Tc3,
           Ws1, bs1, Ws2, bs2):
    x_st = jnp.concatenate([x1, x2], axis=0)
    s1, d1 = _pad_edges(edge_index1, 0)
    s2, d2 = _pad_edges(edge_index2, 1)
    srcs = jnp.stack([s1, s2])
    dsts = jnp.stack([d1, d2])

    ch, bk, nbi = CHUNK_W
    srcs_w = srcs.reshape(NC, NT, nbi, bk, ch)
    dsts_w = dsts.reshape(NC, NT, nbi, bk, ch)
    b1r, b2r, b3r = b1[None, :], b2[None, :], b3[None, :]
    t1f = T1.reshape(128, 128 * 128)
    t2f = T2.reshape(64, 64 * 64)
    t3f = T3.reshape(32, 32 * 32)
    tb1t, tb2t, tb3t = Tb1.T, Tb2.T, Tb3.T
    tc1r, tc2r, tc3r = Tc1.T, Tc2.T, Tc3.T
    bs1r, bs2r = bs1[None, :], bs2[None, :]

    h1_raw = _tc_k1a(x_st, W1)
    deg_st = _deg_kernel(dsts_w)

    h1 = _tc_k1b(h1_raw, deg_st)
    a1 = _agg_kernel(128)(h1, srcs_w, dsts_w)
    f1, h2, cs1 = _tc_layer(a1, deg_st, b1r, W2, 128, 64)
    a2 = _agg_kernel(64)(h2, srcs_w, dsts_w)
    p1 = _tc_attn1(f1, cs1, Wa1, 128)
    f2, h3, cs2 = _tc_layer(a2, deg_st, b2r, W3, 64, 32)
    a3 = _agg_kernel(32)(h3, srcs_w, dsts_w)
    p2 = _tc_attn1(f2, cs2, Wa2, 64)
    f3, cs3 = _tc_last(a3, deg_st, b3r, 32)
    p3 = _tc_attn1(f3, cs3, Wa3, 32)

    o1, o2, o3 = _tc_ntn_a(p1, p2, p3, t1f, t2f, t3f)
    m1 = o1.reshape(128, 128)
    m2 = o2.reshape(64, 64)
    m3 = o3.reshape(32, 32)
    return _tc_ntn_b(m1, m2, m3, p1, p2, p3, tb1t, tb2t, tb3t,
                     tc1r, tc2r, tc3r, Ws1, bs1r, Ws2, bs2r)
